# Initial kernel scaffold; baseline (speedup 1.0000x reference)
import jax
import jax.numpy as jnp
from jax.experimental import pallas as pl


def kernel(input_ids, token_table, position_table):
    raise NotImplementedError("write your pallas kernel here")


# speedup vs baseline: 2.1457x; 2.1457x over previous
import functools

import jax
import jax.numpy as jnp
from jax import lax
from jax.experimental import pallas as pl
from jax.experimental.pallas import tpu as pltpu
from jax.experimental.pallas import tpu_sc as plsc

_D = 64
_SEQ = 200
_LANES = 16

_NC = 2
_NS = 16
_NW = _NC * _NS

_C = 512
_IDXROWS = _C // 128


def _emb_body(nrows, ids_hbm, tok_hbm, pos_hbm, out_hbm, idx_v, rows_v, pos_v,
              gsem):
    bpw = nrows // _NW
    nchunk = bpw // _C
    wid = lax.axis_index("s") * _NC + lax.axis_index("c")
    base = wid * bpw
    base_row = wid * (bpw // 128)

    pltpu.sync_copy(pos_hbm, pos_v)

    def chunk_body(g, _):
        row0 = base_row + g * _IDXROWS
        pltpu.sync_copy(ids_hbm.at[pl.ds(row0, _IDXROWS)], idx_v)
        cps = [
            pltpu.async_copy(tok_hbm.at[idx_v.at[j]],
                             rows_v.at[pl.ds(j * 128, 128)], gsem)
            for j in range(_IDXROWS)
        ]
        for cp in cps:
            cp.wait()

        p0 = lax.rem(g * _C, _SEQ)

        def row_body(r, p):
            for s in range(_D // _LANES):
                x = pos_v[p, pl.ds(s * _LANES, _LANES)]
                plsc.addupdate(rows_v.at[r, pl.ds(s * _LANES, _LANES)], x)
            p = p + 1
            return jnp.where(p == _SEQ, 0, p)

        lax.fori_loop(0, _C, row_body, p0)

        pltpu.sync_copy(rows_v, out_hbm.at[pl.ds(base + g * _C, _C)])
        return 0

    lax.fori_loop(0, nchunk, chunk_body, 0)


def _make_lookup(nrows):
    mesh = plsc.VectorSubcoreMesh(core_axis_name="c", subcore_axis_name="s")
    return functools.partial(
        pl.kernel,
        out_type=jax.ShapeDtypeStruct((nrows, _D), jnp.float32),
        mesh=mesh,
        scratch_types=[
            pltpu.VMEM((_IDXROWS, 128), jnp.int32),
            pltpu.VMEM((_C, _D), jnp.float32),
            pltpu.VMEM((_SEQ, _D), jnp.float32),
            pltpu.SemaphoreType.DMA,
        ],
        compiler_params=pltpu.CompilerParams(use_tc_tiling_on_sc=False),
    )(functools.partial(_emb_body, nrows))


def kernel(input_ids, token_table, position_table):
    batch, seq = input_ids.shape
    nrows = batch * seq
    ids = input_ids.astype(jnp.int32).reshape(nrows // 128, 128)
    out = _make_lookup(nrows)(ids, token_table, position_table)
    return out.reshape(batch, seq, _D)
